# baseline (speedup 1.0000x reference)
import functools

import jax
import jax.numpy as jnp
from jax import lax
from jax.experimental import pallas as pl
from jax.experimental.pallas import tpu as pltpu
from jax.experimental.pallas import tpu_sc as plsc

N_DOW = 7
N_MONTH = 12
N_LEAP = 2
N_DECADES = 30
NJ = N_DOW * N_MONTH * N_LEAP * N_DECADES
B = 16384
DIM = 96
ED = 16
DP = 128

NC = 2
NS = 16
NW = NC * NS
BPW = B // NW


def _table_body(dow_emb, month_emb, leap_emb, decade_emb_t, joint_emb_t,
                W1, b1, out_ref):
    f32 = jnp.float32
    j = lax.broadcasted_iota(jnp.int32, (NJ, 1), 0)

    def onehot(col, n):
        return (col == lax.broadcasted_iota(jnp.int32, (NJ, n), 1)).astype(f32)

    def dot00(a, b):
        return lax.dot_general(a, b, (((0,), (0,)), ((), ())),
                               preferred_element_type=f32)

    a_dow = jnp.dot(dow_emb[...], W1[0:ED, :], preferred_element_type=f32)
    a_mon = jnp.dot(month_emb[...], W1[ED:2 * ED, :], preferred_element_type=f32)
    a_leap = jnp.dot(leap_emb[...], W1[2 * ED:3 * ED, :], preferred_element_type=f32)
    a_dec = dot00(decade_emb_t[...], W1[3 * ED:4 * ED, :])

    acc = dot00(joint_emb_t[...], W1[4 * ED:, :])
    acc = acc + jnp.dot(onehot(j // (N_MONTH * N_LEAP * N_DECADES), N_DOW),
                        a_dow, preferred_element_type=f32)
    acc = acc + jnp.dot(onehot((j // (N_LEAP * N_DECADES)) % N_MONTH, N_MONTH),
                        a_mon, preferred_element_type=f32)
    acc = acc + jnp.dot(onehot((j // N_DECADES) % N_LEAP, N_LEAP),
                        a_leap, preferred_element_type=f32)
    acc = acc + jnp.dot(onehot(j % N_DECADES, N_DECADES),
                        a_dec, preferred_element_type=f32)
    out_ref[:, pl.ds(0, DIM)] = acc + b1[...]


def _build_table(dow_emb, month_emb, leap_emb, decade_emb, joint_emb, W1, b1):
    return pl.pallas_call(
        _table_body,
        out_shape=jax.ShapeDtypeStruct((NJ, DP), jnp.float32),
    )(dow_emb, month_emb, leap_emb, decade_emb.T, joint_emb.T,
      W1, b1.reshape(1, DIM))


def _sc_gather_body(table_hbm, dow_hbm, month_hbm, leap_hbm, decade_hbm,
                    out_hbm, idx_v, rows_v, d_v, m_v, l_v, c_v, sem):
    wid = lax.axis_index("s") * NC + lax.axis_index("c")
    base = wid * BPW
    pltpu.sync_copy(dow_hbm.at[pl.ds(base, BPW)], d_v)
    pltpu.sync_copy(month_hbm.at[pl.ds(base, BPW)], m_v)
    pltpu.sync_copy(leap_hbm.at[pl.ds(base, BPW)], l_v)
    pltpu.sync_copy(decade_hbm.at[pl.ds(base, BPW)], c_v)
    for i in range(BPW // 16):
        sl = pl.ds(i * 16, 16)
        idx_v[sl] = ((d_v[sl] * N_MONTH + m_v[sl]) * N_LEAP
                     + l_v[sl]) * N_DECADES + c_v[sl]
    pltpu.async_copy(table_hbm.at[idx_v], rows_v, sem).wait()
    pltpu.sync_copy(rows_v, out_hbm.at[pl.ds(base, BPW)])


@functools.cache
def _sc_gather():
    return functools.partial(
        pl.kernel,
        mesh=plsc.VectorSubcoreMesh(core_axis_name="c", subcore_axis_name="s"),
        out_type=jax.ShapeDtypeStruct((B, DP), jnp.float32),
        scratch_types=[
            pltpu.VMEM((BPW,), jnp.int32),
            pltpu.VMEM((BPW, DP), jnp.float32),
            pltpu.VMEM((BPW,), jnp.int32),
            pltpu.VMEM((BPW,), jnp.int32),
            pltpu.VMEM((BPW,), jnp.int32),
            pltpu.VMEM((BPW,), jnp.int32),
            pltpu.SemaphoreType.DMA,
        ],
    )(_sc_gather_body)


def _mlp2_body(hpre, W2, b2, out_ref):
    f32 = jnp.float32
    h = jax.nn.gelu(hpre[:, pl.ds(0, DIM)])
    tmp = jnp.dot(h, W2[...], preferred_element_type=f32)
    out_ref[...] = tmp.T + b2[...]


def _mlp2(hpre_rows, W2, b2):
    return pl.pallas_call(
        _mlp2_body,
        out_shape=jax.ShapeDtypeStruct((DIM, B), jnp.float32),
        grid=(8,),
        in_specs=[
            pl.BlockSpec((B // 8, DP), lambda i: (i, 0)),
            pl.BlockSpec((DIM, DIM), lambda i: (0, 0)),
            pl.BlockSpec((DIM, 1), lambda i: (0, 0)),
        ],
        out_specs=pl.BlockSpec((DIM, B // 8), lambda i: (0, i)),
    )(hpre_rows, W2, b2.reshape(DIM, 1))


def kernel(dow, month, leap, decade, dow_emb, month_emb, leap_emb, decade_emb,
           joint_emb, W1, b1, W2, b2):
    table = _build_table(dow_emb, month_emb, leap_emb, decade_emb, joint_emb,
                         W1, b1)
    hpre_rows = _sc_gather()(table, dow, month, leap, decade)
    return _mlp2(hpre_rows, W2, b2).T


# speedup vs baseline: 1.0886x; 1.0886x over previous
import functools

import jax
import jax.numpy as jnp
from jax import lax
from jax.experimental import pallas as pl
from jax.experimental.pallas import tpu as pltpu
from jax.experimental.pallas import tpu_sc as plsc

N_DOW = 7
N_MONTH = 12
N_LEAP = 2
N_DECADES = 30
NJ = N_DOW * N_MONTH * N_LEAP * N_DECADES
B = 16384
DIM = 96
ED = 16
DP = 128

NC = 2
NS = 16
NW = NC * NS
BPW = B // NW


NSUB = N_MONTH * N_LEAP * N_DECADES


def _table_body(dow_emb, month_emb, leap_emb, decade_emb_t, joint_emb_t,
                W1, b1, out_ref):
    f32 = jnp.float32
    jj = lax.broadcasted_iota(jnp.int32, (NSUB, 1), 0)

    def onehot(col, n):
        return (col == lax.broadcasted_iota(jnp.int32, (NSUB, n), 1)).astype(f32)

    def dot00(a, b):
        return lax.dot_general(a, b, (((0,), (0,)), ((), ())),
                               preferred_element_type=f32)

    a_dow = jnp.dot(dow_emb[...], W1[0:ED, :], preferred_element_type=f32)
    a_mon = jnp.dot(month_emb[...], W1[ED:2 * ED, :], preferred_element_type=f32)
    a_leap = jnp.dot(leap_emb[...], W1[2 * ED:3 * ED, :], preferred_element_type=f32)
    a_dec = dot00(decade_emb_t[...], W1[3 * ED:4 * ED, :])

    sub = jnp.dot(onehot(jj // (N_LEAP * N_DECADES), N_MONTH),
                  a_mon, preferred_element_type=f32)
    sub = sub + jnp.dot(onehot((jj // N_DECADES) % N_LEAP, N_LEAP),
                        a_leap, preferred_element_type=f32)
    sub = sub + jnp.dot(onehot(jj % N_DECADES, N_DECADES),
                        a_dec, preferred_element_type=f32)
    sub = sub + b1[...]
    small = (a_dow.reshape(N_DOW, 1, DIM)
             + sub.reshape(1, NSUB, DIM)).reshape(NJ, DIM)
    acc = dot00(joint_emb_t[...], W1[4 * ED:, :])
    out_ref[:, pl.ds(0, DIM)] = acc + small


def _build_table(dow_emb, month_emb, leap_emb, decade_emb, joint_emb, W1, b1):
    return pl.pallas_call(
        _table_body,
        out_shape=jax.ShapeDtypeStruct((NJ, DP), jnp.float32),
    )(dow_emb, month_emb, leap_emb, decade_emb.T, joint_emb.T,
      W1, b1.reshape(1, DIM))


def _sc_gather_body(table_hbm, dow_hbm, month_hbm, leap_hbm, decade_hbm,
                    out_hbm, idx_v, rows_v, d_v, m_v, l_v, c_v, sem):
    wid = lax.axis_index("s") * NC + lax.axis_index("c")
    base = wid * BPW
    pltpu.sync_copy(dow_hbm.at[pl.ds(base, BPW)], d_v)
    pltpu.sync_copy(month_hbm.at[pl.ds(base, BPW)], m_v)
    pltpu.sync_copy(leap_hbm.at[pl.ds(base, BPW)], l_v)
    pltpu.sync_copy(decade_hbm.at[pl.ds(base, BPW)], c_v)
    for i in range(BPW // 16):
        sl = pl.ds(i * 16, 16)
        idx_v[sl] = ((d_v[sl] * N_MONTH + m_v[sl]) * N_LEAP
                     + l_v[sl]) * N_DECADES + c_v[sl]
    pltpu.async_copy(table_hbm.at[idx_v], rows_v, sem).wait()
    pltpu.sync_copy(rows_v, out_hbm.at[pl.ds(base, BPW)])


@functools.cache
def _sc_gather():
    return functools.partial(
        pl.kernel,
        mesh=plsc.VectorSubcoreMesh(core_axis_name="c", subcore_axis_name="s"),
        out_type=jax.ShapeDtypeStruct((B, DP), jnp.float32),
        scratch_types=[
            pltpu.VMEM((BPW,), jnp.int32),
            pltpu.VMEM((BPW, DP), jnp.float32),
            pltpu.VMEM((BPW,), jnp.int32),
            pltpu.VMEM((BPW,), jnp.int32),
            pltpu.VMEM((BPW,), jnp.int32),
            pltpu.VMEM((BPW,), jnp.int32),
            pltpu.SemaphoreType.DMA,
        ],
    )(_sc_gather_body)


def _mlp2_body(hpre, W2, b2, out_ref):
    f32 = jnp.float32
    h = jax.nn.gelu(hpre[:, pl.ds(0, DIM)])
    tmp = jnp.dot(h, W2[...], preferred_element_type=f32)
    out_ref[...] = tmp.T + b2[...]


def _mlp2(hpre_rows, W2, b2):
    return pl.pallas_call(
        _mlp2_body,
        out_shape=jax.ShapeDtypeStruct((DIM, B), jnp.float32),
        grid=(8,),
        in_specs=[
            pl.BlockSpec((B // 8, DP), lambda i: (i, 0)),
            pl.BlockSpec((DIM, DIM), lambda i: (0, 0)),
            pl.BlockSpec((DIM, 1), lambda i: (0, 0)),
        ],
        out_specs=pl.BlockSpec((DIM, B // 8), lambda i: (0, i)),
    )(hpre_rows, W2, b2.reshape(DIM, 1))


def kernel(dow, month, leap, decade, dow_emb, month_emb, leap_emb, decade_emb,
           joint_emb, W1, b1, W2, b2):
    table = _build_table(dow_emb, month_emb, leap_emb, decade_emb, joint_emb,
                         W1, b1)
    hpre_rows = _sc_gather()(table, dow, month, leap, decade)
    return _mlp2(hpre_rows, W2, b2).T
